# baseline (speedup 1.0000x reference)
import jax
import jax.numpy as jnp
from jax import lax
from jax.experimental import pallas as pl
from jax.experimental.pallas import tpu as pltpu
from jax.experimental.pallas import tpu_sc as plsc

N_NUM = 13
N_CAT = 26
CARD = 1000
D = 128
B = 4096
N_TOK = N_NUM + N_CAT

_INFO = plsc.get_sparse_core_info()
_NC = _INFO.num_cores
_NS = _INFO.num_subcores
_NW = _NC * _NS
_RPW = B // _NW
_CHUNK = 8
_CODES = _CHUNK * N_CAT
_NBUF = 4
_NSUPER = _RPW // (_CHUNK * _NBUF)


def _sc_body(xcat_hbm, tab_hbm, catg_hbm, xcat_v,
             idx0, idx1, idx2, idx3, gat0, gat1, gat2, gat3,
             sg0, sg1, sg2, sg3, so0, so1, so2, so3):
    wid = lax.axis_index("s") * _NC + lax.axis_index("c")
    base = wid * _RPW

    idxs = [idx0, idx1, idx2, idx3]
    gats = [gat0, gat1, gat2, gat3]
    sgs = [sg0, sg1, sg2, sg3]
    sos = [so0, so1, so2, so3]

    pltpu.sync_copy(xcat_hbm.at[pl.ds(base * N_CAT, _RPW * N_CAT)], xcat_v)

    lanes = lax.iota(jnp.int32, 16)

    def out_slice(c):
        return catg_hbm.at[pl.ds((base + c * _CHUNK) * N_CAT, _CODES)]

    def super_step(s, carry):
        for k in range(_NBUF):
            c = s * _NBUF + k
            b = k
            pc = c - 1
            pb = (k - 1) % _NBUF

            @pl.when(s > 0)
            def _():
                pltpu.make_async_copy(gats[b], out_slice(c), sos[b]).wait()

            def idx_fill(j, carry2, b=b, c=c):
                p = c * _CODES + j * 16 + lanes
                f = lax.rem(p, N_CAT)
                code = xcat_v[pl.ds(c * _CODES + j * 16, 16)]
                idxs[b][pl.ds(j * 16, 16)] = code + f * CARD
                return carry2

            lax.fori_loop(0, _CODES // 16, idx_fill, 0)

            pltpu.async_copy(tab_hbm.at[idxs[b]], gats[b], sgs[b])

            def drain(pc=pc, pb=pb):
                pltpu.make_async_copy(
                    tab_hbm.at[idxs[pb]], gats[pb], sgs[pb]).wait()
                pltpu.async_copy(gats[pb], out_slice(pc), sos[pb])

            if k == 0:
                @pl.when(s > 0)
                def _():
                    drain()
            else:
                drain()
        return carry

    lax.fori_loop(0, _NSUPER, super_step, 0)

    last = _NSUPER * _NBUF - 1
    lb = last % _NBUF
    pltpu.make_async_copy(tab_hbm.at[idxs[lb]], gats[lb], sgs[lb]).wait()
    pltpu.async_copy(gats[lb], out_slice(last), sos[lb])
    for k in range(_NBUF):
        c = last - (_NBUF - 1) + k
        pltpu.make_async_copy(gats[k], out_slice(c), sos[k]).wait()


_BB = 64


def _asm_body(xnum_ref, catg_ref, w_ref, nb_ref, cb_ref, out_ref):
    xn = xnum_ref[...]
    w = w_ref[...]
    nb = nb_ref[...]
    cb = cb_ref[...]
    num = xn[:, :N_NUM, None] * w[None, :N_NUM, :] + nb[None, :N_NUM, :]
    out_ref[:, :N_NUM, :] = num
    cat = catg_ref[...].reshape(_BB, N_CAT, D) + cb[None, :N_CAT, :]
    out_ref[:, N_NUM:, :] = cat


def _assemble(x_num_pad, catg, w_pad, nb_pad, cb_pad):
    grid = B // _BB
    return pl.pallas_call(
        _asm_body,
        grid=(grid,),
        in_specs=[
            pl.BlockSpec((_BB, 16), lambda i: (i, 0)),
            pl.BlockSpec((_BB * N_CAT, D), lambda i: (i, 0)),
            pl.BlockSpec((16, D), lambda i: (0, 0)),
            pl.BlockSpec((16, D), lambda i: (0, 0)),
            pl.BlockSpec((32, D), lambda i: (0, 0)),
        ],
        out_specs=pl.BlockSpec((_BB, N_TOK, D), lambda i: (i, 0, 0)),
        out_shape=jax.ShapeDtypeStruct((B, N_TOK, D), jnp.float32),
    )(x_num_pad, catg, w_pad, nb_pad, cb_pad)


@jax.jit
def _run(x_num_pad, x_cat_flat, w_pad, nb_pad, cat_table, cb_pad):
    mesh = plsc.VectorSubcoreMesh(core_axis_name="c", subcore_axis_name="s")
    gather = pl.kernel(
        _sc_body,
        mesh=mesh,
        out_type=jax.ShapeDtypeStruct((B * N_CAT, D), jnp.float32),
        scratch_types=(
            [pltpu.VMEM((_RPW * N_CAT,), jnp.int32)]
            + [pltpu.VMEM((_CODES,), jnp.int32) for _ in range(_NBUF)]
            + [pltpu.VMEM((_CODES, D), jnp.float32) for _ in range(_NBUF)]
            + [pltpu.SemaphoreType.DMA for _ in range(2 * _NBUF)]
        ),
    )
    catg = gather(x_cat_flat, cat_table)
    return _assemble(x_num_pad, catg, w_pad, nb_pad, cb_pad)


def kernel(x_num, x_cat, num_weight, num_bias, cat_table, cat_bias):
    x_num_pad = jnp.pad(x_num, ((0, 0), (0, 16 - N_NUM)))
    x_cat_flat = x_cat.astype(jnp.int32).reshape(-1)
    w_pad = jnp.pad(num_weight, ((0, 16 - N_NUM), (0, 0)))
    nb_pad = jnp.pad(num_bias, ((0, 16 - N_NUM), (0, 0)))
    cb_pad = jnp.pad(cat_bias, ((0, 32 - N_CAT), (0, 0)))
    return _run(x_num_pad, x_cat_flat, w_pad, nb_pad, cat_table, cb_pad)
